# Initial kernel scaffold; baseline (speedup 1.0000x reference)
import jax
import jax.numpy as jnp
from jax.experimental import pallas as pl


def kernel(hidden_states, pre_ln_gamma, post_ln_gamma, Wq, Wk, Wv, Wo, gate_w, We_gate, We_up, We_down, Ws1, Ws3, Ws2):
    raise NotImplementedError("write your pallas kernel here")


# speedup vs baseline: 1.0251x; 1.0251x over previous
import functools

import jax
import jax.numpy as jnp
from jax.experimental import pallas as pl

B, S, D = 1, 2048, 1024
H, DH = 16, 64
E, K = 8, 2
DFF, DSH = 512, 2048
EPS = 1e-6

BS = 256
NSB = S // BS


def _rms(x, g):
    return x * jax.lax.rsqrt(jnp.mean(x * x, axis=-1, keepdims=True) + EPS) * g


def _qkv_kernel(x_ref, g_ref, wq_ref, wk_ref, wv_ref, q_ref, k_ref, v_ref):
    h = _rms(x_ref[...], g_ref[...])
    q_ref[...] = jnp.dot(h, wq_ref[...], preferred_element_type=jnp.float32)
    k_ref[...] = jnp.dot(h, wk_ref[...], preferred_element_type=jnp.float32)
    v_ref[...] = jnp.dot(h, wv_ref[...], preferred_element_type=jnp.float32)


def _qkv(x, gamma, Wq, Wk, Wv):
    g2 = gamma.reshape(1, D)
    return pl.pallas_call(
        _qkv_kernel,
        grid=(NSB,),
        in_specs=[
            pl.BlockSpec((BS, D), lambda i: (i, 0)),
            pl.BlockSpec((1, D), lambda i: (0, 0)),
            pl.BlockSpec((D, H * DH), lambda i: (0, 0)),
            pl.BlockSpec((D, H * DH), lambda i: (0, 0)),
            pl.BlockSpec((D, H * DH), lambda i: (0, 0)),
        ],
        out_specs=[
            pl.BlockSpec((BS, H * DH), lambda i: (i, 0)),
            pl.BlockSpec((BS, H * DH), lambda i: (i, 0)),
            pl.BlockSpec((BS, H * DH), lambda i: (i, 0)),
        ],
        out_shape=[jax.ShapeDtypeStruct((S, H * DH), jnp.float32)] * 3,
    )(x, g2, Wq, Wk, Wv)


def _attn_kernel(q_ref, k_ref, v_ref, o_ref, *, scale):
    i = pl.program_id(1)
    q = q_ref[0]
    k = k_ref[0]
    v = v_ref[0]
    s = jax.lax.dot_general(q, k, (((1,), (1,)), ((), ())),
                            preferred_element_type=jnp.float32) * scale
    rows = jax.lax.broadcasted_iota(jnp.int32, (BS, S), 0) + i * BS
    cols = jax.lax.broadcasted_iota(jnp.int32, (BS, S), 1)
    s = jnp.where(cols <= rows, s, jnp.float32(-1e9))
    m = jnp.max(s, axis=-1, keepdims=True)
    p = jnp.exp(s - m)
    p = p / jnp.sum(p, axis=-1, keepdims=True)
    o_ref[0] = jnp.dot(p, v, preferred_element_type=jnp.float32)


def _attention(q3, k3, v3):
    scale = 1.0 / float(DH) ** 0.5
    return pl.pallas_call(
        functools.partial(_attn_kernel, scale=scale),
        grid=(H, NSB),
        in_specs=[
            pl.BlockSpec((1, BS, DH), lambda h, i: (h, i, 0)),
            pl.BlockSpec((1, S, DH), lambda h, i: (h, 0, 0)),
            pl.BlockSpec((1, S, DH), lambda h, i: (h, 0, 0)),
        ],
        out_specs=pl.BlockSpec((1, BS, DH), lambda h, i: (h, i, 0)),
        out_shape=jax.ShapeDtypeStruct((H, S, DH), jnp.float32),
    )(q3, k3, v3)


def _proj_router_kernel(a_ref, wo_ref, res_ref, g_ref, gw_ref,
                        h_ref, x2_ref, wfull_ref):
    attn = jnp.dot(a_ref[0], wo_ref[0], preferred_element_type=jnp.float32)
    for hh in range(1, H):
        attn += jnp.dot(a_ref[hh], wo_ref[hh],
                        preferred_element_type=jnp.float32)
    hstate = res_ref[...] + attn
    h_ref[...] = hstate
    x2 = _rms(hstate, g_ref[...])
    x2_ref[...] = x2
    logits = jnp.dot(x2, gw_ref[...], preferred_element_type=jnp.float32)
    m = jnp.max(logits, axis=-1, keepdims=True)
    p = jnp.exp(logits - m)
    p = p / jnp.sum(p, axis=-1, keepdims=True)
    idx = jax.lax.broadcasted_iota(jnp.int32, (BS, E), 1)
    m1 = jnp.max(p, axis=-1, keepdims=True)
    i1 = jnp.min(jnp.where(p == m1, idx, E), axis=-1, keepdims=True)
    p2 = jnp.where(idx == i1, -jnp.inf, p)
    m2 = jnp.max(p2, axis=-1, keepdims=True)
    i2 = jnp.min(jnp.where(p2 == m2, idx, E), axis=-1, keepdims=True)
    tot = m1 + m2
    wfull_ref[...] = jnp.where(idx == i1, m1 / tot, 0.0) + \
        jnp.where(idx == i2, m2 / tot, 0.0)


def _proj_router(attn, Wo, residual, gamma, gate_w):
    g2 = gamma.reshape(1, D)
    return pl.pallas_call(
        _proj_router_kernel,
        grid=(NSB,),
        in_specs=[
            pl.BlockSpec((H, BS, DH), lambda i: (0, i, 0)),
            pl.BlockSpec((H, DH, D), lambda i: (0, 0, 0)),
            pl.BlockSpec((BS, D), lambda i: (i, 0)),
            pl.BlockSpec((1, D), lambda i: (0, 0)),
            pl.BlockSpec((D, E), lambda i: (0, 0)),
        ],
        out_specs=[
            pl.BlockSpec((BS, D), lambda i: (i, 0)),
            pl.BlockSpec((BS, D), lambda i: (i, 0)),
            pl.BlockSpec((BS, E), lambda i: (i, 0)),
        ],
        out_shape=[
            jax.ShapeDtypeStruct((S, D), jnp.float32),
            jax.ShapeDtypeStruct((S, D), jnp.float32),
            jax.ShapeDtypeStruct((S, E), jnp.float32),
        ],
    )(attn, Wo.reshape(H, DH, D), residual, g2, gate_w)


def _moe_kernel(x_ref, wg_ref, wu_ref, wd_ref, w_ref, o_ref):
    e = pl.program_id(0)

    @pl.when(e == 0)
    def _():
        o_ref[...] = jnp.zeros_like(o_ref)

    x = x_ref[...]
    g = jnp.dot(x, wg_ref[0], preferred_element_type=jnp.float32)
    u = jnp.dot(x, wu_ref[0], preferred_element_type=jnp.float32)
    a = g * jax.lax.logistic(g) * u
    d = jnp.dot(a, wd_ref[0], preferred_element_type=jnp.float32)
    o_ref[...] += w_ref[0] * d


def _moe(x2, We_gate, We_up, We_down, w_full):
    wt = w_full.T.reshape(E, S, 1)
    return pl.pallas_call(
        _moe_kernel,
        grid=(E,),
        in_specs=[
            pl.BlockSpec((S, D), lambda e: (0, 0)),
            pl.BlockSpec((1, D, DFF), lambda e: (e, 0, 0)),
            pl.BlockSpec((1, D, DFF), lambda e: (e, 0, 0)),
            pl.BlockSpec((1, DFF, D), lambda e: (e, 0, 0)),
            pl.BlockSpec((1, S, 1), lambda e: (e, 0, 0)),
        ],
        out_specs=pl.BlockSpec((S, D), lambda e: (0, 0)),
        out_shape=jax.ShapeDtypeStruct((S, D), jnp.float32),
    )(x2, We_gate, We_up, We_down, wt)


def _shared_kernel(x_ref, w1_ref, w3_ref, w2_ref, h_ref, moe_ref, o_ref):
    x = x_ref[...]
    g = jnp.dot(x, w1_ref[...], preferred_element_type=jnp.float32)
    u = jnp.dot(x, w3_ref[...], preferred_element_type=jnp.float32)
    a = g * jax.lax.logistic(g) * u
    sh = jnp.dot(a, w2_ref[...], preferred_element_type=jnp.float32)
    o_ref[...] = h_ref[...] + moe_ref[...] + sh


def _shared(x2, Ws1, Ws3, Ws2, hstate, moe_out):
    return pl.pallas_call(
        _shared_kernel,
        grid=(NSB,),
        in_specs=[
            pl.BlockSpec((BS, D), lambda i: (i, 0)),
            pl.BlockSpec((D, DSH), lambda i: (0, 0)),
            pl.BlockSpec((D, DSH), lambda i: (0, 0)),
            pl.BlockSpec((DSH, D), lambda i: (0, 0)),
            pl.BlockSpec((BS, D), lambda i: (i, 0)),
            pl.BlockSpec((BS, D), lambda i: (i, 0)),
        ],
        out_specs=pl.BlockSpec((BS, D), lambda i: (i, 0)),
        out_shape=jax.ShapeDtypeStruct((S, D), jnp.float32),
    )(x2, Ws1, Ws3, Ws2, hstate, moe_out)


def kernel(hidden_states, pre_ln_gamma, post_ln_gamma, Wq, Wk, Wv, Wo,
           gate_w, We_gate, We_up, We_down, Ws1, Ws3, Ws2):
    x = hidden_states.reshape(S, D)
    q, k, v = _qkv(x, pre_ln_gamma, Wq, Wk, Wv)
    q3 = q.reshape(S, H, DH).swapaxes(0, 1)
    k3 = k.reshape(S, H, DH).swapaxes(0, 1)
    v3 = v.reshape(S, H, DH).swapaxes(0, 1)
    attn = _attention(q3, k3, v3)
    hstate, x2, w_full = _proj_router(attn, Wo, x, post_ln_gamma, gate_w)
    moe_out = _moe(x2, We_gate, We_up, We_down, w_full)
    out = _shared(x2, Ws1, Ws3, Ws2, hstate, moe_out)
    return out.reshape(B, S, D)


# speedup vs baseline: 1.1217x; 1.0942x over previous
import functools

import jax
import jax.numpy as jnp
from jax.experimental import pallas as pl

B, S, D = 1, 2048, 1024
H, DH = 16, 64
E, K = 8, 2
DFF, DSH = 512, 2048
EPS = 1e-6

BS = 256
NSB = S // BS


def _rms(x, g):
    return x * jax.lax.rsqrt(jnp.mean(x * x, axis=-1, keepdims=True) + EPS) * g


def _qkv_kernel(x_ref, g_ref, wq_ref, wk_ref, wv_ref, q_ref, k_ref, v_ref):
    h = _rms(x_ref[...], g_ref[...]).astype(jnp.bfloat16)
    q_ref[...] = jnp.dot(h, wq_ref[...],
                         preferred_element_type=jnp.float32).astype(jnp.bfloat16)
    k_ref[...] = jnp.dot(h, wk_ref[...],
                         preferred_element_type=jnp.float32).astype(jnp.bfloat16)
    v_ref[...] = jnp.dot(h, wv_ref[...],
                         preferred_element_type=jnp.float32).astype(jnp.bfloat16)


def _qkv(x, gamma, Wq, Wk, Wv):
    g2 = gamma.reshape(1, D)
    return pl.pallas_call(
        _qkv_kernel,
        grid=(NSB,),
        in_specs=[
            pl.BlockSpec((BS, D), lambda i: (i, 0)),
            pl.BlockSpec((1, D), lambda i: (0, 0)),
            pl.BlockSpec((D, H * DH), lambda i: (0, 0)),
            pl.BlockSpec((D, H * DH), lambda i: (0, 0)),
            pl.BlockSpec((D, H * DH), lambda i: (0, 0)),
        ],
        out_specs=[
            pl.BlockSpec((BS, H * DH), lambda i: (i, 0)),
            pl.BlockSpec((BS, H * DH), lambda i: (i, 0)),
            pl.BlockSpec((BS, H * DH), lambda i: (i, 0)),
        ],
        out_shape=[jax.ShapeDtypeStruct((S, H * DH), jnp.bfloat16)] * 3,
    )(x, g2, Wq, Wk, Wv)


def _attn_kernel(q_ref, k_ref, v_ref, o_ref, *, scale):
    i = pl.program_id(1)
    q = q_ref[0]
    k = k_ref[0]
    v = v_ref[0]
    s = jax.lax.dot_general(q, k, (((1,), (1,)), ((), ())),
                            preferred_element_type=jnp.float32) * scale
    rows = jax.lax.broadcasted_iota(jnp.int32, (BS, S), 0) + i * BS
    cols = jax.lax.broadcasted_iota(jnp.int32, (BS, S), 1)
    s = jnp.where(cols <= rows, s, jnp.float32(-1e9))
    m = jnp.max(s, axis=-1, keepdims=True)
    p = jnp.exp(s - m)
    p = (p / jnp.sum(p, axis=-1, keepdims=True)).astype(jnp.bfloat16)
    o_ref[0] = jnp.dot(p, v,
                       preferred_element_type=jnp.float32).astype(jnp.bfloat16)


def _attention(q3, k3, v3):
    scale = 1.0 / float(DH) ** 0.5
    return pl.pallas_call(
        functools.partial(_attn_kernel, scale=scale),
        grid=(H, NSB),
        in_specs=[
            pl.BlockSpec((1, BS, DH), lambda h, i: (h, i, 0)),
            pl.BlockSpec((1, S, DH), lambda h, i: (h, 0, 0)),
            pl.BlockSpec((1, S, DH), lambda h, i: (h, 0, 0)),
        ],
        out_specs=pl.BlockSpec((1, BS, DH), lambda h, i: (h, i, 0)),
        out_shape=jax.ShapeDtypeStruct((H, S, DH), jnp.bfloat16),
    )(q3, k3, v3)


def _proj_router_kernel(a_ref, wo_ref, res_ref, g_ref, gw_ref,
                        h_ref, x2_ref, wfull_ref):
    attn = jnp.dot(a_ref[0], wo_ref[0], preferred_element_type=jnp.float32)
    for hh in range(1, H):
        attn += jnp.dot(a_ref[hh], wo_ref[hh],
                        preferred_element_type=jnp.float32)
    hstate = res_ref[...] + attn
    h_ref[...] = hstate
    x2 = _rms(hstate, g_ref[...])
    x2_ref[...] = x2.astype(jnp.bfloat16)
    logits = jnp.dot(x2, gw_ref[...], preferred_element_type=jnp.float32)
    m = jnp.max(logits, axis=-1, keepdims=True)
    p = jnp.exp(logits - m)
    p = p / jnp.sum(p, axis=-1, keepdims=True)
    idx = jax.lax.broadcasted_iota(jnp.int32, (BS, E), 1)
    m1 = jnp.max(p, axis=-1, keepdims=True)
    i1 = jnp.min(jnp.where(p == m1, idx, E), axis=-1, keepdims=True)
    p2 = jnp.where(idx == i1, -jnp.inf, p)
    m2 = jnp.max(p2, axis=-1, keepdims=True)
    i2 = jnp.min(jnp.where(p2 == m2, idx, E), axis=-1, keepdims=True)
    tot = m1 + m2
    wfull_ref[...] = jnp.where(idx == i1, m1 / tot, 0.0) + \
        jnp.where(idx == i2, m2 / tot, 0.0)


def _proj_router(attn, Wo, residual, gamma, gate_w):
    g2 = gamma.reshape(1, D)
    return pl.pallas_call(
        _proj_router_kernel,
        grid=(NSB,),
        in_specs=[
            pl.BlockSpec((H, BS, DH), lambda i: (0, i, 0)),
            pl.BlockSpec((H, DH, D), lambda i: (0, 0, 0)),
            pl.BlockSpec((BS, D), lambda i: (i, 0)),
            pl.BlockSpec((1, D), lambda i: (0, 0)),
            pl.BlockSpec((D, E), lambda i: (0, 0)),
        ],
        out_specs=[
            pl.BlockSpec((BS, D), lambda i: (i, 0)),
            pl.BlockSpec((BS, D), lambda i: (i, 0)),
            pl.BlockSpec((BS, E), lambda i: (i, 0)),
        ],
        out_shape=[
            jax.ShapeDtypeStruct((S, D), jnp.float32),
            jax.ShapeDtypeStruct((S, D), jnp.bfloat16),
            jax.ShapeDtypeStruct((S, E), jnp.float32),
        ],
    )(attn, Wo.reshape(H, DH, D), residual, g2, gate_w)


def _moe_kernel(x_ref, wg_ref, wu_ref, wd_ref, w_ref, o_ref):
    e = pl.program_id(0)

    @pl.when(e == 0)
    def _():
        o_ref[...] = jnp.zeros_like(o_ref)

    x = x_ref[...]
    g = jnp.dot(x, wg_ref[0], preferred_element_type=jnp.float32)
    u = jnp.dot(x, wu_ref[0], preferred_element_type=jnp.float32)
    a = (g * jax.lax.logistic(g) * u).astype(jnp.bfloat16)
    d = jnp.dot(a, wd_ref[0], preferred_element_type=jnp.float32)
    o_ref[...] += w_ref[0] * d


def _moe(x2, We_gate, We_up, We_down, w_full):
    wt = w_full.T.reshape(E, S, 1)
    return pl.pallas_call(
        _moe_kernel,
        grid=(E,),
        in_specs=[
            pl.BlockSpec((S, D), lambda e: (0, 0)),
            pl.BlockSpec((1, D, DFF), lambda e: (e, 0, 0)),
            pl.BlockSpec((1, D, DFF), lambda e: (e, 0, 0)),
            pl.BlockSpec((1, DFF, D), lambda e: (e, 0, 0)),
            pl.BlockSpec((1, S, 1), lambda e: (e, 0, 0)),
        ],
        out_specs=pl.BlockSpec((S, D), lambda e: (0, 0)),
        out_shape=jax.ShapeDtypeStruct((S, D), jnp.float32),
    )(x2, We_gate, We_up, We_down, wt)


def _shared_kernel(x_ref, w1_ref, w3_ref, w2_ref, h_ref, moe_ref, o_ref):
    x = x_ref[...]
    g = jnp.dot(x, w1_ref[...], preferred_element_type=jnp.float32)
    u = jnp.dot(x, w3_ref[...], preferred_element_type=jnp.float32)
    a = (g * jax.lax.logistic(g) * u).astype(jnp.bfloat16)
    sh = jnp.dot(a, w2_ref[...], preferred_element_type=jnp.float32)
    o_ref[...] = h_ref[...] + moe_ref[...] + sh


def _shared(x2, Ws1, Ws3, Ws2, hstate, moe_out):
    return pl.pallas_call(
        _shared_kernel,
        grid=(NSB,),
        in_specs=[
            pl.BlockSpec((BS, D), lambda i: (i, 0)),
            pl.BlockSpec((D, DSH), lambda i: (0, 0)),
            pl.BlockSpec((D, DSH), lambda i: (0, 0)),
            pl.BlockSpec((DSH, D), lambda i: (0, 0)),
            pl.BlockSpec((BS, D), lambda i: (i, 0)),
            pl.BlockSpec((BS, D), lambda i: (i, 0)),
        ],
        out_specs=pl.BlockSpec((BS, D), lambda i: (i, 0)),
        out_shape=jax.ShapeDtypeStruct((S, D), jnp.float32),
    )(x2, Ws1, Ws3, Ws2, hstate, moe_out)


def kernel(hidden_states, pre_ln_gamma, post_ln_gamma, Wq, Wk, Wv, Wo,
           gate_w, We_gate, We_up, We_down, Ws1, Ws3, Ws2):
    bf = jnp.bfloat16
    x = hidden_states.reshape(S, D)
    q, k, v = _qkv(x, pre_ln_gamma, Wq.astype(bf), Wk.astype(bf), Wv.astype(bf))
    q3 = q.reshape(S, H, DH).swapaxes(0, 1)
    k3 = k.reshape(S, H, DH).swapaxes(0, 1)
    v3 = v.reshape(S, H, DH).swapaxes(0, 1)
    attn = _attention(q3, k3, v3)
    hstate, x2, w_full = _proj_router(attn, Wo.astype(bf), x, post_ln_gamma, gate_w)
    moe_out = _moe(x2, We_gate.astype(bf), We_up.astype(bf), We_down.astype(bf), w_full)
    out = _shared(x2, Ws1.astype(bf), Ws3.astype(bf), Ws2.astype(bf), hstate, moe_out)
    return out.reshape(B, S, D)


# speedup vs baseline: 1.2445x; 1.1095x over previous
import functools

import jax
import jax.numpy as jnp
from jax.experimental import pallas as pl

B, S, D = 1, 2048, 1024
H, DH = 16, 64
E, K = 8, 2
DFF, DSH = 512, 2048
EPS = 1e-6

BS = 256
NSB = S // BS


def _rms(x, g):
    return x * jax.lax.rsqrt(jnp.mean(x * x, axis=-1, keepdims=True) + EPS) * g


def _qkv_kernel(x_ref, g_ref, wq_ref, wk_ref, wv_ref, q_ref, k_ref, v_ref):
    h = _rms(x_ref[...], g_ref[...]).astype(jnp.bfloat16)
    q_ref[...] = jnp.dot(h, wq_ref[...],
                         preferred_element_type=jnp.float32).astype(jnp.bfloat16)
    k_ref[...] = jnp.dot(h, wk_ref[...],
                         preferred_element_type=jnp.float32).astype(jnp.bfloat16)
    v_ref[...] = jnp.dot(h, wv_ref[...],
                         preferred_element_type=jnp.float32).astype(jnp.bfloat16)


def _qkv(x, gamma, Wq, Wk, Wv):
    g2 = gamma.reshape(1, D)
    return pl.pallas_call(
        _qkv_kernel,
        grid=(NSB,),
        in_specs=[
            pl.BlockSpec((BS, D), lambda i: (i, 0)),
            pl.BlockSpec((1, D), lambda i: (0, 0)),
            pl.BlockSpec((D, H * DH), lambda i: (0, 0)),
            pl.BlockSpec((D, H * DH), lambda i: (0, 0)),
            pl.BlockSpec((D, H * DH), lambda i: (0, 0)),
        ],
        out_specs=[
            pl.BlockSpec((BS, H * DH), lambda i: (i, 0)),
            pl.BlockSpec((BS, H * DH), lambda i: (i, 0)),
            pl.BlockSpec((BS, H * DH), lambda i: (i, 0)),
        ],
        out_shape=[jax.ShapeDtypeStruct((S, H * DH), jnp.bfloat16)] * 3,
    )(x, g2, Wq, Wk, Wv)


BQ = 512
NQB = S // BQ


def _attn_kernel(q_ref, k_ref, v_ref, o_ref, *, scale):
    i = pl.program_id(1)
    q = q_ref[0]
    rows = jax.lax.broadcasted_iota(jnp.int32, (BQ, BQ), 0) + i * BQ

    def body(j, carry):
        m, l, acc = carry
        base = pl.multiple_of(j * BQ, BQ)
        k = k_ref[0, pl.ds(base, BQ), :]
        v = v_ref[0, pl.ds(base, BQ), :]
        s = jax.lax.dot_general(q, k, (((1,), (1,)), ((), ())),
                                preferred_element_type=jnp.float32) * scale
        cols = jax.lax.broadcasted_iota(jnp.int32, (BQ, BQ), 1) + j * BQ
        s = jnp.where(cols <= rows, s, jnp.float32(-1e30))
        m_new = jnp.maximum(m, jnp.max(s, axis=-1, keepdims=True))
        alpha = jnp.exp(m - m_new)
        p = jnp.exp(s - m_new)
        l = l * alpha + jnp.sum(p, axis=-1, keepdims=True)
        acc = acc * alpha + jnp.dot(p.astype(jnp.bfloat16), v,
                                    preferred_element_type=jnp.float32)
        return m_new, l, acc

    m0 = jnp.full((BQ, 1), -1e30, jnp.float32)
    l0 = jnp.zeros((BQ, 1), jnp.float32)
    a0 = jnp.zeros((BQ, DH), jnp.float32)
    m, l, acc = jax.lax.fori_loop(0, i + 1, body, (m0, l0, a0))
    o_ref[0] = (acc * (1.0 / l)).astype(jnp.bfloat16)


def _attention(q3, k3, v3):
    scale = 1.0 / float(DH) ** 0.5
    return pl.pallas_call(
        functools.partial(_attn_kernel, scale=scale),
        grid=(H, NQB),
        in_specs=[
            pl.BlockSpec((1, BQ, DH), lambda h, i: (h, i, 0)),
            pl.BlockSpec((1, S, DH), lambda h, i: (h, 0, 0)),
            pl.BlockSpec((1, S, DH), lambda h, i: (h, 0, 0)),
        ],
        out_specs=pl.BlockSpec((1, BQ, DH), lambda h, i: (h, i, 0)),
        out_shape=jax.ShapeDtypeStruct((H, S, DH), jnp.bfloat16),
    )(q3, k3, v3)


def _proj_router_kernel(a_ref, wo_ref, res_ref, g_ref, gw_ref,
                        h_ref, x2_ref, wfull_ref):
    attn = jnp.dot(a_ref[0], wo_ref[0], preferred_element_type=jnp.float32)
    for hh in range(1, H):
        attn += jnp.dot(a_ref[hh], wo_ref[hh],
                        preferred_element_type=jnp.float32)
    hstate = res_ref[...] + attn
    h_ref[...] = hstate
    x2 = _rms(hstate, g_ref[...])
    x2_ref[...] = x2.astype(jnp.bfloat16)
    logits = jnp.dot(x2, gw_ref[...], preferred_element_type=jnp.float32)
    m = jnp.max(logits, axis=-1, keepdims=True)
    p = jnp.exp(logits - m)
    p = p / jnp.sum(p, axis=-1, keepdims=True)
    idx = jax.lax.broadcasted_iota(jnp.int32, (BS, E), 1)
    m1 = jnp.max(p, axis=-1, keepdims=True)
    i1 = jnp.min(jnp.where(p == m1, idx, E), axis=-1, keepdims=True)
    p2 = jnp.where(idx == i1, -jnp.inf, p)
    m2 = jnp.max(p2, axis=-1, keepdims=True)
    i2 = jnp.min(jnp.where(p2 == m2, idx, E), axis=-1, keepdims=True)
    tot = m1 + m2
    wfull_ref[...] = jnp.where(idx == i1, m1 / tot, 0.0) + \
        jnp.where(idx == i2, m2 / tot, 0.0)


def _proj_router(attn, Wo, residual, gamma, gate_w):
    g2 = gamma.reshape(1, D)
    return pl.pallas_call(
        _proj_router_kernel,
        grid=(NSB,),
        in_specs=[
            pl.BlockSpec((H, BS, DH), lambda i: (0, i, 0)),
            pl.BlockSpec((H, DH, D), lambda i: (0, 0, 0)),
            pl.BlockSpec((BS, D), lambda i: (i, 0)),
            pl.BlockSpec((1, D), lambda i: (0, 0)),
            pl.BlockSpec((D, E), lambda i: (0, 0)),
        ],
        out_specs=[
            pl.BlockSpec((BS, D), lambda i: (i, 0)),
            pl.BlockSpec((BS, D), lambda i: (i, 0)),
            pl.BlockSpec((BS, E), lambda i: (i, 0)),
        ],
        out_shape=[
            jax.ShapeDtypeStruct((S, D), jnp.float32),
            jax.ShapeDtypeStruct((S, D), jnp.bfloat16),
            jax.ShapeDtypeStruct((S, E), jnp.float32),
        ],
    )(attn, Wo.reshape(H, DH, D), residual, g2, gate_w)


def _moe_kernel(x_ref, wg_ref, wu_ref, wd_ref, w_ref, o_ref):
    e = pl.program_id(0)

    @pl.when(e == 0)
    def _():
        o_ref[...] = jnp.zeros_like(o_ref)

    x = x_ref[...]
    g = jnp.dot(x, wg_ref[0], preferred_element_type=jnp.float32)
    u = jnp.dot(x, wu_ref[0], preferred_element_type=jnp.float32)
    a = (g * jax.lax.logistic(g) * u).astype(jnp.bfloat16)
    d = jnp.dot(a, wd_ref[0], preferred_element_type=jnp.float32)
    o_ref[...] += w_ref[0] * d


def _moe(x2, We_gate, We_up, We_down, w_full):
    wt = w_full.T.reshape(E, S, 1)
    return pl.pallas_call(
        _moe_kernel,
        grid=(E,),
        in_specs=[
            pl.BlockSpec((S, D), lambda e: (0, 0)),
            pl.BlockSpec((1, D, DFF), lambda e: (e, 0, 0)),
            pl.BlockSpec((1, D, DFF), lambda e: (e, 0, 0)),
            pl.BlockSpec((1, DFF, D), lambda e: (e, 0, 0)),
            pl.BlockSpec((1, S, 1), lambda e: (e, 0, 0)),
        ],
        out_specs=pl.BlockSpec((S, D), lambda e: (0, 0)),
        out_shape=jax.ShapeDtypeStruct((S, D), jnp.float32),
    )(x2, We_gate, We_up, We_down, wt)


def _shared_kernel(x_ref, w1_ref, w3_ref, w2_ref, h_ref, moe_ref, o_ref):
    x = x_ref[...]
    g = jnp.dot(x, w1_ref[...], preferred_element_type=jnp.float32)
    u = jnp.dot(x, w3_ref[...], preferred_element_type=jnp.float32)
    a = (g * jax.lax.logistic(g) * u).astype(jnp.bfloat16)
    sh = jnp.dot(a, w2_ref[...], preferred_element_type=jnp.float32)
    o_ref[...] = h_ref[...] + moe_ref[...] + sh


def _shared(x2, Ws1, Ws3, Ws2, hstate, moe_out):
    return pl.pallas_call(
        _shared_kernel,
        grid=(NSB,),
        in_specs=[
            pl.BlockSpec((BS, D), lambda i: (i, 0)),
            pl.BlockSpec((D, DSH), lambda i: (0, 0)),
            pl.BlockSpec((D, DSH), lambda i: (0, 0)),
            pl.BlockSpec((DSH, D), lambda i: (0, 0)),
            pl.BlockSpec((BS, D), lambda i: (i, 0)),
            pl.BlockSpec((BS, D), lambda i: (i, 0)),
        ],
        out_specs=pl.BlockSpec((BS, D), lambda i: (i, 0)),
        out_shape=jax.ShapeDtypeStruct((S, D), jnp.float32),
    )(x2, Ws1, Ws3, Ws2, hstate, moe_out)


def kernel(hidden_states, pre_ln_gamma, post_ln_gamma, Wq, Wk, Wv, Wo,
           gate_w, We_gate, We_up, We_down, Ws1, Ws3, Ws2):
    bf = jnp.bfloat16
    x = hidden_states.reshape(S, D)
    q, k, v = _qkv(x, pre_ln_gamma, Wq.astype(bf), Wk.astype(bf), Wv.astype(bf))
    q3 = q.reshape(S, H, DH).swapaxes(0, 1)
    k3 = k.reshape(S, H, DH).swapaxes(0, 1)
    v3 = v.reshape(S, H, DH).swapaxes(0, 1)
    attn = _attention(q3, k3, v3)
    hstate, x2, w_full = _proj_router(attn, Wo.astype(bf), x, post_ln_gamma, gate_w)
    moe_out = _moe(x2, We_gate.astype(bf), We_up.astype(bf), We_down.astype(bf), w_full)
    out = _shared(x2, Ws1.astype(bf), Ws3.astype(bf), Ws2.astype(bf), hstate, moe_out)
    return out.reshape(B, S, D)


# speedup vs baseline: 1.4221x; 1.1428x over previous
import functools

import jax
import jax.numpy as jnp
from jax.experimental import pallas as pl

B, S, D = 1, 2048, 1024
H, DH = 16, 64
E, K = 8, 2
DFF, DSH = 512, 2048
EPS = 1e-6

BS = 256
NSB = S // BS
H2 = H // 2
DP = 2 * DH
BQ = 512
NQB = S // BQ


def _rms(x, g):
    return x * jax.lax.rsqrt(jnp.mean(x * x, axis=-1, keepdims=True) + EPS) * g


def _qkv_kernel(x_ref, g_ref, wq_ref, wk_ref, wv_ref, q_ref, k_ref, v_ref):
    h = _rms(x_ref[...], g_ref[...]).astype(jnp.bfloat16)
    q = jnp.dot(h, wq_ref[...],
                preferred_element_type=jnp.float32).astype(jnp.bfloat16)
    k = jnp.dot(h, wk_ref[...],
                preferred_element_type=jnp.float32).astype(jnp.bfloat16)
    v = jnp.dot(h, wv_ref[...],
                preferred_element_type=jnp.float32).astype(jnp.bfloat16)
    q_ref[...] = q.reshape(BS, H2, DP).swapaxes(0, 1)
    k_ref[...] = k.reshape(BS, H2, DP).swapaxes(0, 1)
    v_ref[...] = v.reshape(BS, H2, DP).swapaxes(0, 1)


def _qkv(x, gamma, Wq, Wk, Wv):
    g2 = gamma.reshape(1, D)
    return pl.pallas_call(
        _qkv_kernel,
        grid=(NSB,),
        in_specs=[
            pl.BlockSpec((BS, D), lambda i: (i, 0)),
            pl.BlockSpec((1, D), lambda i: (0, 0)),
            pl.BlockSpec((D, H * DH), lambda i: (0, 0)),
            pl.BlockSpec((D, H * DH), lambda i: (0, 0)),
            pl.BlockSpec((D, H * DH), lambda i: (0, 0)),
        ],
        out_specs=[
            pl.BlockSpec((H2, BS, DP), lambda i: (0, i, 0)),
            pl.BlockSpec((H2, BS, DP), lambda i: (0, i, 0)),
            pl.BlockSpec((H2, BS, DP), lambda i: (0, i, 0)),
        ],
        out_shape=[jax.ShapeDtypeStruct((H2, S, DP), jnp.bfloat16)] * 3,
    )(x, g2, Wq, Wk, Wv)


def _attn_kernel(q_ref, k_ref, v_ref, o_ref, *, scale):
    i = pl.program_id(1)
    q2 = q_ref[0]
    lane = jax.lax.broadcasted_iota(jnp.int32, (BQ, DP), 1)
    is_a = lane < DH
    zero = jnp.zeros((), jnp.bfloat16)
    qa = jnp.where(is_a, q2, zero)
    qb = jnp.where(is_a, zero, q2)
    rows = jax.lax.broadcasted_iota(jnp.int32, (BQ, BQ), 0) + i * BQ

    def body(j, carry):
        ma, la, aa, mb, lb, ab = carry
        base = pl.multiple_of(j * BQ, BQ)
        k2 = k_ref[0, pl.ds(base, BQ), :]
        v2 = v_ref[0, pl.ds(base, BQ), :]
        cols = jax.lax.broadcasted_iota(jnp.int32, (BQ, BQ), 1) + j * BQ
        causal = cols <= rows

        def one(qh, m, l, acc):
            s = jax.lax.dot_general(qh, k2, (((1,), (1,)), ((), ())),
                                    preferred_element_type=jnp.float32) * scale
            s = jnp.where(causal, s, jnp.float32(-1e30))
            m_new = jnp.maximum(m, jnp.max(s, axis=-1, keepdims=True))
            alpha = jnp.exp(m - m_new)
            p = jnp.exp(s - m_new)
            l = l * alpha + jnp.sum(p, axis=-1, keepdims=True)
            acc = acc * alpha + jnp.dot(p.astype(jnp.bfloat16), v2,
                                        preferred_element_type=jnp.float32)
            return m_new, l, acc

        ma, la, aa = one(qa, ma, la, aa)
        mb, lb, ab = one(qb, mb, lb, ab)
        return ma, la, aa, mb, lb, ab

    m0 = jnp.full((BQ, 1), -1e30, jnp.float32)
    l0 = jnp.zeros((BQ, 1), jnp.float32)
    a0 = jnp.zeros((BQ, DP), jnp.float32)
    ma, la, aa, mb, lb, ab = jax.lax.fori_loop(
        0, i + 1, body, (m0, l0, a0, m0, l0, a0))
    oa = aa * (1.0 / la)
    ob = ab * (1.0 / lb)
    o_ref[0] = jnp.where(is_a, oa, ob).astype(jnp.bfloat16)


def _attention(q3, k3, v3):
    scale = 1.0 / float(DH) ** 0.5
    return pl.pallas_call(
        functools.partial(_attn_kernel, scale=scale),
        grid=(H2, NQB),
        in_specs=[
            pl.BlockSpec((1, BQ, DP), lambda h, i: (h, i, 0)),
            pl.BlockSpec((1, S, DP), lambda h, i: (h, 0, 0)),
            pl.BlockSpec((1, S, DP), lambda h, i: (h, 0, 0)),
        ],
        out_specs=pl.BlockSpec((1, BQ, DP), lambda h, i: (h, i, 0)),
        out_shape=jax.ShapeDtypeStruct((H2, S, DP), jnp.bfloat16),
    )(q3, k3, v3)


def _proj_router_kernel(a_ref, wo_ref, res_ref, g_ref, gw_ref,
                        h_ref, x2_ref, wfull_ref):
    attn = jnp.dot(a_ref[0], wo_ref[0], preferred_element_type=jnp.float32)
    for hh in range(1, H2):
        attn += jnp.dot(a_ref[hh], wo_ref[hh],
                        preferred_element_type=jnp.float32)
    hstate = res_ref[...] + attn
    h_ref[...] = hstate
    x2 = _rms(hstate, g_ref[...])
    x2_ref[...] = x2.astype(jnp.bfloat16)
    logits = jnp.dot(x2, gw_ref[...], preferred_element_type=jnp.float32)
    m = jnp.max(logits, axis=-1, keepdims=True)
    p = jnp.exp(logits - m)
    p = p / jnp.sum(p, axis=-1, keepdims=True)
    idx = jax.lax.broadcasted_iota(jnp.int32, (BS, E), 1)
    m1 = jnp.max(p, axis=-1, keepdims=True)
    i1 = jnp.min(jnp.where(p == m1, idx, E), axis=-1, keepdims=True)
    p2 = jnp.where(idx == i1, -jnp.inf, p)
    m2 = jnp.max(p2, axis=-1, keepdims=True)
    i2 = jnp.min(jnp.where(p2 == m2, idx, E), axis=-1, keepdims=True)
    tot = m1 + m2
    wfull_ref[...] = jnp.where(idx == i1, m1 / tot, 0.0) + \
        jnp.where(idx == i2, m2 / tot, 0.0)


def _proj_router(attn, Wo, residual, gamma, gate_w):
    g2 = gamma.reshape(1, D)
    return pl.pallas_call(
        _proj_router_kernel,
        grid=(NSB,),
        in_specs=[
            pl.BlockSpec((H2, BS, DP), lambda i: (0, i, 0)),
            pl.BlockSpec((H2, DP, D), lambda i: (0, 0, 0)),
            pl.BlockSpec((BS, D), lambda i: (i, 0)),
            pl.BlockSpec((1, D), lambda i: (0, 0)),
            pl.BlockSpec((D, E), lambda i: (0, 0)),
        ],
        out_specs=[
            pl.BlockSpec((BS, D), lambda i: (i, 0)),
            pl.BlockSpec((BS, D), lambda i: (i, 0)),
            pl.BlockSpec((BS, E), lambda i: (i, 0)),
        ],
        out_shape=[
            jax.ShapeDtypeStruct((S, D), jnp.float32),
            jax.ShapeDtypeStruct((S, D), jnp.bfloat16),
            jax.ShapeDtypeStruct((S, E), jnp.float32),
        ],
    )(attn, Wo.reshape(H2, DP, D), residual, g2, gate_w)


def _moe_kernel(x_ref, wg_ref, wu_ref, wd_ref, w_ref, o_ref):
    e = pl.program_id(0)

    @pl.when(e == 0)
    def _():
        o_ref[...] = jnp.zeros_like(o_ref)

    x = x_ref[...]
    g = jnp.dot(x, wg_ref[0], preferred_element_type=jnp.float32)
    u = jnp.dot(x, wu_ref[0], preferred_element_type=jnp.float32)
    a = (g * jax.lax.logistic(g) * u).astype(jnp.bfloat16)
    d = jnp.dot(a, wd_ref[0], preferred_element_type=jnp.float32)
    o_ref[...] += w_ref[0] * d


def _moe(x2, We_gate, We_up, We_down, w_full):
    wt = w_full.T.reshape(E, S, 1)
    return pl.pallas_call(
        _moe_kernel,
        grid=(E,),
        in_specs=[
            pl.BlockSpec((S, D), lambda e: (0, 0)),
            pl.BlockSpec((1, D, DFF), lambda e: (e, 0, 0)),
            pl.BlockSpec((1, D, DFF), lambda e: (e, 0, 0)),
            pl.BlockSpec((1, DFF, D), lambda e: (e, 0, 0)),
            pl.BlockSpec((1, S, 1), lambda e: (e, 0, 0)),
        ],
        out_specs=pl.BlockSpec((S, D), lambda e: (0, 0)),
        out_shape=jax.ShapeDtypeStruct((S, D), jnp.float32),
    )(x2, We_gate, We_up, We_down, wt)


def _shared_kernel(x_ref, w1_ref, w3_ref, w2_ref, h_ref, moe_ref, o_ref):
    x = x_ref[...]
    g = jnp.dot(x, w1_ref[...], preferred_element_type=jnp.float32)
    u = jnp.dot(x, w3_ref[...], preferred_element_type=jnp.float32)
    a = (g * jax.lax.logistic(g) * u).astype(jnp.bfloat16)
    sh = jnp.dot(a, w2_ref[...], preferred_element_type=jnp.float32)
    o_ref[...] = h_ref[...] + moe_ref[...] + sh


def _shared(x2, Ws1, Ws3, Ws2, hstate, moe_out):
    return pl.pallas_call(
        _shared_kernel,
        grid=(NSB,),
        in_specs=[
            pl.BlockSpec((BS, D), lambda i: (i, 0)),
            pl.BlockSpec((D, DSH), lambda i: (0, 0)),
            pl.BlockSpec((D, DSH), lambda i: (0, 0)),
            pl.BlockSpec((DSH, D), lambda i: (0, 0)),
            pl.BlockSpec((BS, D), lambda i: (i, 0)),
            pl.BlockSpec((BS, D), lambda i: (i, 0)),
        ],
        out_specs=pl.BlockSpec((BS, D), lambda i: (i, 0)),
        out_shape=jax.ShapeDtypeStruct((S, D), jnp.float32),
    )(x2, Ws1, Ws3, Ws2, hstate, moe_out)


def kernel(hidden_states, pre_ln_gamma, post_ln_gamma, Wq, Wk, Wv, Wo,
           gate_w, We_gate, We_up, We_down, Ws1, Ws3, Ws2):
    bf = jnp.bfloat16
    x = hidden_states.reshape(S, D)
    q3, k3, v3 = _qkv(x, pre_ln_gamma, Wq.astype(bf), Wk.astype(bf),
                      Wv.astype(bf))
    attn = _attention(q3, k3, v3)
    hstate, x2, w_full = _proj_router(attn, Wo.astype(bf), x, post_ln_gamma,
                                      gate_w)
    moe_out = _moe(x2, We_gate.astype(bf), We_up.astype(bf),
                   We_down.astype(bf), w_full)
    out = _shared(x2, Ws1.astype(bf), Ws3.astype(bf), Ws2.astype(bf),
                  hstate, moe_out)
    return out.reshape(B, S, D)


# speedup vs baseline: 1.6213x; 1.1401x over previous
import functools

import jax
import jax.numpy as jnp
from jax.experimental import pallas as pl

B, S, D = 1, 2048, 1024
H, DH = 16, 64
E, K = 8, 2
DFF, DSH = 512, 2048
EPS = 1e-6

BS = 256
NSB = S // BS
H2 = H // 2
DP = 2 * DH
BQ = 512
NQB = S // BQ


def _rms(x, g):
    return x * jax.lax.rsqrt(jnp.mean(x * x, axis=-1, keepdims=True) + EPS) * g


def _qkv_kernel(x_ref, g_ref, wq_ref, wk_ref, wv_ref, q_ref, k_ref, v_ref):
    h = _rms(x_ref[...], g_ref[...]).astype(jnp.bfloat16)
    q = jnp.dot(h, wq_ref[...].astype(jnp.bfloat16),
                preferred_element_type=jnp.float32).astype(jnp.bfloat16)
    k = jnp.dot(h, wk_ref[...].astype(jnp.bfloat16),
                preferred_element_type=jnp.float32).astype(jnp.bfloat16)
    v = jnp.dot(h, wv_ref[...].astype(jnp.bfloat16),
                preferred_element_type=jnp.float32).astype(jnp.bfloat16)
    q_ref[...] = q.reshape(BS, H2, DP).swapaxes(0, 1)
    k_ref[...] = k.reshape(BS, H2, DP).swapaxes(0, 1)
    v_ref[...] = v.reshape(BS, H2, DP).swapaxes(0, 1)


def _qkv(x, gamma, Wq, Wk, Wv):
    g2 = gamma.reshape(1, D)
    return pl.pallas_call(
        _qkv_kernel,
        grid=(NSB,),
        in_specs=[
            pl.BlockSpec((BS, D), lambda i: (i, 0)),
            pl.BlockSpec((1, D), lambda i: (0, 0)),
            pl.BlockSpec((D, H * DH), lambda i: (0, 0)),
            pl.BlockSpec((D, H * DH), lambda i: (0, 0)),
            pl.BlockSpec((D, H * DH), lambda i: (0, 0)),
        ],
        out_specs=[
            pl.BlockSpec((H2, BS, DP), lambda i: (0, i, 0)),
            pl.BlockSpec((H2, BS, DP), lambda i: (0, i, 0)),
            pl.BlockSpec((H2, BS, DP), lambda i: (0, i, 0)),
        ],
        out_shape=[jax.ShapeDtypeStruct((H2, S, DP), jnp.bfloat16)] * 3,
    )(x, g2, Wq, Wk, Wv)


def _attn_kernel(q_ref, k_ref, v_ref, o_ref, *, scale):
    i = pl.program_id(1)
    q2 = q_ref[0]
    lane = jax.lax.broadcasted_iota(jnp.int32, (BQ, DP), 1)
    is_a = lane < DH
    zero = jnp.zeros((), jnp.bfloat16)
    qa = jnp.where(is_a, q2, zero)
    qb = jnp.where(is_a, zero, q2)
    rows = jax.lax.broadcasted_iota(jnp.int32, (BQ, BQ), 0) + i * BQ

    def body(j, carry):
        ma, la, aa, mb, lb, ab = carry
        base = pl.multiple_of(j * BQ, BQ)
        k2 = k_ref[0, pl.ds(base, BQ), :]
        v2 = v_ref[0, pl.ds(base, BQ), :]
        cols = jax.lax.broadcasted_iota(jnp.int32, (BQ, BQ), 1) + j * BQ
        causal = cols <= rows

        def one(qh, m, l, acc):
            s = jax.lax.dot_general(qh, k2, (((1,), (1,)), ((), ())),
                                    preferred_element_type=jnp.float32) * scale
            s = jnp.where(causal, s, jnp.float32(-1e30))
            m_new = jnp.maximum(m, jnp.max(s, axis=-1, keepdims=True))
            alpha = jnp.exp(m - m_new)
            p = jnp.exp(s - m_new)
            l = l * alpha + jnp.sum(p, axis=-1, keepdims=True)
            acc = acc * alpha + jnp.dot(p.astype(jnp.bfloat16), v2,
                                        preferred_element_type=jnp.float32)
            return m_new, l, acc

        ma, la, aa = one(qa, ma, la, aa)
        mb, lb, ab = one(qb, mb, lb, ab)
        return ma, la, aa, mb, lb, ab

    m0 = jnp.full((BQ, 1), -1e30, jnp.float32)
    l0 = jnp.zeros((BQ, 1), jnp.float32)
    a0 = jnp.zeros((BQ, DP), jnp.float32)
    ma, la, aa, mb, lb, ab = jax.lax.fori_loop(
        0, i + 1, body, (m0, l0, a0, m0, l0, a0))
    oa = aa * (1.0 / la)
    ob = ab * (1.0 / lb)
    o_ref[0] = jnp.where(is_a, oa, ob).astype(jnp.bfloat16)


def _attention(q3, k3, v3):
    scale = 1.0 / float(DH) ** 0.5
    return pl.pallas_call(
        functools.partial(_attn_kernel, scale=scale),
        grid=(H2, NQB),
        in_specs=[
            pl.BlockSpec((1, BQ, DP), lambda h, i: (h, i, 0)),
            pl.BlockSpec((1, S, DP), lambda h, i: (h, 0, 0)),
            pl.BlockSpec((1, S, DP), lambda h, i: (h, 0, 0)),
        ],
        out_specs=pl.BlockSpec((1, BQ, DP), lambda h, i: (h, i, 0)),
        out_shape=jax.ShapeDtypeStruct((H2, S, DP), jnp.bfloat16),
    )(q3, k3, v3)


def _proj_router_kernel(a_ref, wo_ref, res_ref, g_ref, gw_ref,
                        h_ref, x2_ref, wfull_ref):
    attn = jnp.dot(a_ref[0], wo_ref[0].astype(jnp.bfloat16),
                   preferred_element_type=jnp.float32)
    for hh in range(1, H2):
        attn += jnp.dot(a_ref[hh], wo_ref[hh].astype(jnp.bfloat16),
                        preferred_element_type=jnp.float32)
    hstate = res_ref[...] + attn
    h_ref[...] = hstate
    x2 = _rms(hstate, g_ref[...])
    x2_ref[...] = x2.astype(jnp.bfloat16)
    logits = jnp.dot(x2, gw_ref[...], preferred_element_type=jnp.float32)
    m = jnp.max(logits, axis=-1, keepdims=True)
    p = jnp.exp(logits - m)
    p = p / jnp.sum(p, axis=-1, keepdims=True)
    idx = jax.lax.broadcasted_iota(jnp.int32, (BS, E), 1)
    m1 = jnp.max(p, axis=-1, keepdims=True)
    i1 = jnp.min(jnp.where(p == m1, idx, E), axis=-1, keepdims=True)
    p2 = jnp.where(idx == i1, -jnp.inf, p)
    m2 = jnp.max(p2, axis=-1, keepdims=True)
    i2 = jnp.min(jnp.where(p2 == m2, idx, E), axis=-1, keepdims=True)
    tot = m1 + m2
    wfull_ref[...] = jnp.where(idx == i1, m1 / tot, 0.0) + \
        jnp.where(idx == i2, m2 / tot, 0.0)


def _proj_router(attn, Wo, residual, gamma, gate_w):
    g2 = gamma.reshape(1, D)
    return pl.pallas_call(
        _proj_router_kernel,
        grid=(NSB,),
        in_specs=[
            pl.BlockSpec((H2, BS, DP), lambda i: (0, i, 0)),
            pl.BlockSpec((H2, DP, D), lambda i: (0, 0, 0)),
            pl.BlockSpec((BS, D), lambda i: (i, 0)),
            pl.BlockSpec((1, D), lambda i: (0, 0)),
            pl.BlockSpec((D, E), lambda i: (0, 0)),
        ],
        out_specs=[
            pl.BlockSpec((BS, D), lambda i: (i, 0)),
            pl.BlockSpec((BS, D), lambda i: (i, 0)),
            pl.BlockSpec((BS, E), lambda i: (i, 0)),
        ],
        out_shape=[
            jax.ShapeDtypeStruct((S, D), jnp.float32),
            jax.ShapeDtypeStruct((S, D), jnp.bfloat16),
            jax.ShapeDtypeStruct((S, E), jnp.float32),
        ],
    )(attn, Wo.reshape(H2, DP, D), residual, g2, gate_w)


def _moe_kernel(x_ref, wg_ref, wu_ref, wd_ref, w_ref, o_ref):
    e = pl.program_id(0)

    @pl.when(e == 0)
    def _():
        o_ref[...] = jnp.zeros_like(o_ref)

    x = x_ref[...]
    g = jnp.dot(x, wg_ref[0].astype(jnp.bfloat16),
                preferred_element_type=jnp.float32)
    u = jnp.dot(x, wu_ref[0].astype(jnp.bfloat16),
                preferred_element_type=jnp.float32)
    a = (g * jax.lax.logistic(g) * u).astype(jnp.bfloat16)
    d = jnp.dot(a, wd_ref[0].astype(jnp.bfloat16),
                preferred_element_type=jnp.float32)
    o_ref[...] += w_ref[0] * d


def _moe(x2, We_gate, We_up, We_down, w_full):
    wt = w_full.T.reshape(E, S, 1)
    return pl.pallas_call(
        _moe_kernel,
        grid=(E,),
        in_specs=[
            pl.BlockSpec((S, D), lambda e: (0, 0)),
            pl.BlockSpec((1, D, DFF), lambda e: (e, 0, 0)),
            pl.BlockSpec((1, D, DFF), lambda e: (e, 0, 0)),
            pl.BlockSpec((1, DFF, D), lambda e: (e, 0, 0)),
            pl.BlockSpec((1, S, 1), lambda e: (e, 0, 0)),
        ],
        out_specs=pl.BlockSpec((S, D), lambda e: (0, 0)),
        out_shape=jax.ShapeDtypeStruct((S, D), jnp.float32),
    )(x2, We_gate, We_up, We_down, wt)


def _shared_kernel(x_ref, w1_ref, w3_ref, w2_ref, h_ref, moe_ref, o_ref):
    x = x_ref[...]
    g = jnp.dot(x, w1_ref[...].astype(jnp.bfloat16),
                preferred_element_type=jnp.float32)
    u = jnp.dot(x, w3_ref[...].astype(jnp.bfloat16),
                preferred_element_type=jnp.float32)
    a = (g * jax.lax.logistic(g) * u).astype(jnp.bfloat16)
    sh = jnp.dot(a, w2_ref[...].astype(jnp.bfloat16),
                preferred_element_type=jnp.float32)
    o_ref[...] = h_ref[...] + moe_ref[...] + sh


def _shared(x2, Ws1, Ws3, Ws2, hstate, moe_out):
    return pl.pallas_call(
        _shared_kernel,
        grid=(NSB,),
        in_specs=[
            pl.BlockSpec((BS, D), lambda i: (i, 0)),
            pl.BlockSpec((D, DSH), lambda i: (0, 0)),
            pl.BlockSpec((D, DSH), lambda i: (0, 0)),
            pl.BlockSpec((DSH, D), lambda i: (0, 0)),
            pl.BlockSpec((BS, D), lambda i: (i, 0)),
            pl.BlockSpec((BS, D), lambda i: (i, 0)),
        ],
        out_specs=pl.BlockSpec((BS, D), lambda i: (i, 0)),
        out_shape=jax.ShapeDtypeStruct((S, D), jnp.float32),
    )(x2, Ws1, Ws3, Ws2, hstate, moe_out)


def kernel(hidden_states, pre_ln_gamma, post_ln_gamma, Wq, Wk, Wv, Wo,
           gate_w, We_gate, We_up, We_down, Ws1, Ws3, Ws2):
    x = hidden_states.reshape(S, D)
    q3, k3, v3 = _qkv(x, pre_ln_gamma, Wq, Wk, Wv)
    attn = _attention(q3, k3, v3)
    hstate, x2, w_full = _proj_router(attn, Wo, x, post_ln_gamma, gate_w)
    moe_out = _moe(x2, We_gate, We_up, We_down, w_full)
    out = _shared(x2, Ws1, Ws3, Ws2, hstate, moe_out)
    return out.reshape(B, S, D)
